# Initial kernel scaffold; baseline (speedup 1.0000x reference)
import jax
import jax.numpy as jnp
from jax.experimental import pallas as pl


def kernel(text_embeddings, triples_batch, W_gat, att_src, att_dst, bias_gat, W_out, b_out):
    raise NotImplementedError("write your pallas kernel here")


# speedup vs baseline: 22.1068x; 22.1068x over previous
import functools

import jax
import jax.numpy as jnp
from jax.experimental import pallas as pl
from jax.experimental.pallas import tpu as pltpu

B, L, D = 4, 2048, 768
T = 8192
H, C = 4, 256
HC = H * C
N = L

BN = 256
BD = 256
BS = 512
ND = N // BD
NS = N // BS


def _lin_body(emb_ref, wgat_ref, attsrc_ref, attdst_ref,
              xlin_ref, asrcT_ref, adst_ref):
    emb = emb_ref[0]
    xl = jnp.dot(emb, wgat_ref[...], preferred_element_type=jnp.float32)
    xlin_ref[0] = xl
    asrcT_ref[0, 4:8, :] = jnp.zeros((4, BN), jnp.float32)
    for h in range(H):
        xh = xl[:, h * C:(h + 1) * C]
        ats = attsrc_ref[0, h * C:(h + 1) * C][None, :]
        atd = attdst_ref[0, h * C:(h + 1) * C][None, :]
        r = jax.lax.dot_general(ats, xh, (((1,), (1,)), ((), ())),
                                preferred_element_type=jnp.float32)
        asrcT_ref[0, h:h + 1, :] = r
        cdst = jax.lax.dot_general(xh, atd, (((1,), (1,)), ((), ())),
                                   preferred_element_type=jnp.float32)
        adst_ref[0, :, h:h + 1] = cdst


def _lin_call(emb, W_gat, att_src_f, att_dst_f):
    return pl.pallas_call(
        _lin_body,
        grid=(B, N // BN),
        in_specs=[
            pl.BlockSpec((1, BN, D), lambda b, n: (b, n, 0)),
            pl.BlockSpec((D, HC), lambda b, n: (0, 0)),
            pl.BlockSpec((1, HC), lambda b, n: (0, 0)),
            pl.BlockSpec((1, HC), lambda b, n: (0, 0)),
        ],
        out_specs=[
            pl.BlockSpec((1, BN, HC), lambda b, n: (b, n, 0)),
            pl.BlockSpec((1, 8, BN), lambda b, n: (b, 0, n)),
            pl.BlockSpec((1, BN, H), lambda b, n: (b, n, 0)),
        ],
        out_shape=[
            jax.ShapeDtypeStruct((B, N, HC), jnp.float32),
            jax.ShapeDtypeStruct((B, 8, N), jnp.float32),
            jax.ShapeDtypeStruct((B, N, H), jnp.float32),
        ],
        compiler_params=pltpu.CompilerParams(
            dimension_semantics=("parallel", "parallel")),
    )(emb, W_gat, att_src_f, att_dst_f)


def _gat_body(asrcT_ref, adst_ref, cnt_ref, x_ref, emb_ref,
              wout_ref, bgat_ref, bout_ref, out_ref, acc_ref, den_ref):
    s = pl.program_id(2)
    d = pl.program_id(1)

    @pl.when(s == 0)
    def _():
        acc_ref[...] = jnp.zeros_like(acc_ref)
        den_ref[...] = jnp.zeros_like(den_ref)

    cnt = cnt_ref[0]
    rows = d * BD + jax.lax.broadcasted_iota(jnp.int32, (BD, BS), 0)
    cols = s * BS + jax.lax.broadcasted_iota(jnp.int32, (BD, BS), 1)
    cntp = cnt + (rows == cols).astype(jnp.float32)

    for h in range(H):
        asrc_row = asrcT_ref[0, h, pl.ds(s * BS, BS)][None, :]
        gmax = jnp.max(asrcT_ref[0, h, :])
        adst_col = adst_ref[0, :, h:h + 1]
        alpha = adst_col + asrc_row
        alpha = jnp.where(alpha >= 0, alpha, 0.2 * alpha)
        m = gmax + adst_col
        m = jnp.where(m >= 0, m, 0.2 * m)
        w = jnp.exp(alpha - m) * cntp
        den_ref[:, h:h + 1] += jnp.sum(w, axis=1, keepdims=True)
        xh = x_ref[0, :, h * C:(h + 1) * C]
        acc_ref[:, h * C:(h + 1) * C] += jnp.dot(
            w, xh, preferred_element_type=jnp.float32)

    @pl.when(s == NS - 1)
    def _():
        parts = [acc_ref[:, h * C:(h + 1) * C] / den_ref[:, h:h + 1]
                 for h in range(H)]
        xgat = jnp.concatenate(parts, axis=1) + bgat_ref[0][None, :]
        o = jnp.dot(emb_ref[0], wout_ref[:D, :],
                    preferred_element_type=jnp.float32)
        o += jnp.dot(xgat, wout_ref[D:, :],
                     preferred_element_type=jnp.float32)
        out_ref[0] = o + bout_ref[0][None, :]


def _gat_call(asrcT, adst, cnt, xlin, emb, W_out, bgat_f, bout_f):
    return pl.pallas_call(
        _gat_body,
        grid=(B, ND, NS),
        in_specs=[
            pl.BlockSpec((1, 8, N), lambda b, d, s: (b, 0, 0)),
            pl.BlockSpec((1, BD, H), lambda b, d, s: (b, d, 0)),
            pl.BlockSpec((1, BD, BS), lambda b, d, s: (b, d, s)),
            pl.BlockSpec((1, BS, HC), lambda b, d, s: (b, s, 0)),
            pl.BlockSpec((1, BD, D), lambda b, d, s: (b, d, 0)),
            pl.BlockSpec((D + HC, D), lambda b, d, s: (0, 0)),
            pl.BlockSpec((1, HC), lambda b, d, s: (0, 0)),
            pl.BlockSpec((1, D), lambda b, d, s: (0, 0)),
        ],
        out_specs=pl.BlockSpec((1, BD, D), lambda b, d, s: (b, d, 0)),
        out_shape=jax.ShapeDtypeStruct((B, N, D), jnp.float32),
        scratch_shapes=[
            pltpu.VMEM((BD, HC), jnp.float32),
            pltpu.VMEM((BD, H), jnp.float32),
        ],
        compiler_params=pltpu.CompilerParams(
            dimension_semantics=("parallel", "parallel", "arbitrary")),
    )(asrcT, adst, cnt, xlin, emb, W_out, bgat_f, bout_f)


def _count_matrix(triples_batch):
    h = triples_batch[:, :, 0]
    t = triples_batch[:, :, 2]
    dst = jnp.concatenate([t, h], axis=1)
    src = jnp.concatenate([h, t], axis=1)
    bidx = jnp.broadcast_to(jnp.arange(B, dtype=jnp.int32)[:, None], dst.shape)
    cnt = jnp.zeros((B, N, N), jnp.float32)
    cnt = cnt.at[bidx, dst, src].add(1.0)
    return cnt


def kernel(text_embeddings, triples_batch, W_gat, att_src, att_dst,
           bias_gat, W_out, b_out):
    att_src_f = att_src.reshape(1, HC)
    att_dst_f = att_dst.reshape(1, HC)
    bgat_f = bias_gat.reshape(1, HC)
    bout_f = b_out.reshape(1, D)

    cnt = _count_matrix(triples_batch)
    xlin, asrcT, adst = _lin_call(text_embeddings, W_gat, att_src_f, att_dst_f)
    out = _gat_call(asrcT, adst, cnt, xlin, text_embeddings, W_out,
                    bgat_f, bout_f)
    return out
